# baseline (speedup 1.0000x reference)
import jax
import jax.numpy as jnp
from jax.experimental import pallas as pl
from jax.experimental.pallas import tpu as pltpu

_PAD = 1e-9
_W = 512
_HO = 256
_B = 8


def _pool_body(x_ref, s1_ref, s0_ref, s2_ref, o_ref, sc_ref):
    for k in range(4):
        sc_ref[k] = x_ref[:, 128 * k:128 * (k + 1)]

    io = jax.lax.broadcasted_iota(jnp.int32, (_HO, _W), 0)
    jj = jax.lax.broadcasted_iota(jnp.int32, (_HO, _W), 1)
    cond_e = 2 * io + jj == _W
    cond_o = 2 * io + 1 + jj == _W
    jo = jax.lax.broadcasted_iota(jnp.int32, (_HO, _HO), 1)
    iom = jax.lax.broadcasted_iota(jnp.int32, (_HO, _HO), 0)

    for b in range(_B):
        base = b * _W
        xe = jnp.concatenate(
            [sc_ref[pl.ds(k, 1), pl.ds(base, _HO, 2), :][0]
             for k in range(4)], axis=1)
        xo = jnp.concatenate(
            [sc_ref[pl.ds(k, 1), pl.ds(base + 1, _HO, 2), :][0]
             for k in range(4)], axis=1)

        xe = jnp.where(cond_e, _PAD, xe)
        xo = jnp.where(cond_o, _PAD, xo)

        a = jnp.concatenate(
            [jnp.full((1, _W), _PAD, jnp.float32), xo[:-1, :]], axis=0)

        m1 = jnp.maximum(xe, xo)
        m0 = jnp.maximum(a, m1)
        m2 = jnp.maximum(a, xe)

        out = jnp.dot(m0, s0_ref[...], preferred_element_type=jnp.float32)
        out = jnp.maximum(out, jnp.dot(m1, s1_ref[...],
                                       preferred_element_type=jnp.float32))
        out = jnp.maximum(out, jnp.dot(m2, s2_ref[...],
                                       preferred_element_type=jnp.float32))

        out = jnp.where(jo == 0, jnp.maximum(out, _PAD), out)

        o_ref[b * _HO:(b + 1) * _HO, :] = jnp.where(iom + jo < _HO, out, 0.0)


@jax.jit
def kernel(x):
    shape_bac = x.shape[:-2]
    n = 1
    for d in shape_bac:
        n *= d
    xf = x.reshape(n * _W, _W)

    c = jnp.arange(_W, dtype=jnp.int32)[:, None]
    oj2 = 2 * jnp.arange(_HO, dtype=jnp.int32)[None, :]
    s1 = (c == oj2 - 1).astype(jnp.float32)
    s0 = (c == oj2).astype(jnp.float32)
    s2 = (c == oj2 + 1).astype(jnp.float32)

    sel_spec = pl.BlockSpec((_W, _HO), lambda i: (0, 0))
    out = pl.pallas_call(
        _pool_body,
        grid=(n // _B,),
        in_specs=[
            pl.BlockSpec((_B * _W, _W), lambda i: (i, 0)),
            sel_spec, sel_spec, sel_spec,
        ],
        out_specs=pl.BlockSpec((_B * _HO, _HO), lambda i: (i, 0)),
        out_shape=jax.ShapeDtypeStruct((n * _HO, _HO), jnp.float32),
        scratch_shapes=[pltpu.VMEM((4, _B * _W, 128), jnp.float32)],
        compiler_params=pltpu.CompilerParams(
            dimension_semantics=("parallel",),
        ),
        name="hex_max_pool",
    )(xf, s1, s0, s2)

    return out.reshape(*shape_bac, _HO, _HO)


# speedup vs baseline: 1.0404x; 1.0404x over previous
import jax
import jax.numpy as jnp
from jax.experimental import pallas as pl
from jax.experimental.pallas import tpu as pltpu

_PAD = 1e-9
_W = 512
_HO = 256
_B = 8


def _pool_body(x_ref, s1_ref, s0_ref, s2_ref, o_ref, sc_ref):
    for k in range(4):
        sc_ref[k] = x_ref[:, 128 * k:128 * (k + 1)]

    io = jax.lax.broadcasted_iota(jnp.int32, (_HO, _W), 0)
    jj = jax.lax.broadcasted_iota(jnp.int32, (_HO, _W), 1)
    cond_e = 2 * io + jj == _W
    cond_o = 2 * io + 1 + jj == _W
    jo = jax.lax.broadcasted_iota(jnp.int32, (_HO, _HO), 1)
    iom = jax.lax.broadcasted_iota(jnp.int32, (_HO, _HO), 0)

    for b in range(_B):
        base = b * _W
        xe = jnp.concatenate(
            [sc_ref[pl.ds(k, 1), pl.ds(base, _HO, 2), :][0]
             for k in range(4)], axis=1)
        xo = jnp.concatenate(
            [sc_ref[pl.ds(k, 1), pl.ds(base + 1, _HO, 2), :][0]
             for k in range(4)], axis=1)

        xe = jnp.where(cond_e, _PAD, xe)
        xo = jnp.where(cond_o, _PAD, xo)

        a = jnp.concatenate(
            [jnp.full((1, _W), _PAD, jnp.float32), xo[:-1, :]], axis=0)

        m1 = jnp.maximum(xe, xo)
        m0 = jnp.maximum(a, m1)
        m2 = jnp.maximum(a, xe)

        out = jnp.dot(m0, s0_ref[...], preferred_element_type=jnp.float32)
        out = jnp.maximum(out, jnp.dot(m1, s1_ref[...],
                                       preferred_element_type=jnp.float32))
        out = jnp.maximum(out, jnp.dot(m2, s2_ref[...],
                                       preferred_element_type=jnp.float32))

        out = jnp.where(jo == 0, jnp.maximum(out, _PAD), out)

        o_ref[b * _HO:(b + 1) * _HO, :] = jnp.where(iom + jo < _HO, out, 0.0)


@jax.jit
def kernel(x):
    shape_bac = x.shape[:-2]
    n = 1
    for d in shape_bac:
        n *= d
    xf = x.reshape(n * _W, _W)

    c = jnp.arange(_W, dtype=jnp.int32)[:, None]
    oj2 = 2 * jnp.arange(_HO, dtype=jnp.int32)[None, :]
    s1 = (c == oj2 - 1).astype(jnp.float32)
    s0 = (c == oj2).astype(jnp.float32)
    s2 = (c == oj2 + 1).astype(jnp.float32)

    sel_spec = pl.BlockSpec((_W, _HO), lambda i: (0, 0))
    out = pl.pallas_call(
        _pool_body,
        grid=(n // _B,),
        in_specs=[
            pl.BlockSpec((_B * _W, _W), lambda i: (i, 0)),
            sel_spec, sel_spec, sel_spec,
        ],
        out_specs=pl.BlockSpec((_B * _HO, _HO), lambda i: (i, 0)),
        out_shape=jax.ShapeDtypeStruct((n * _HO, _HO), jnp.float32),
        scratch_shapes=[pltpu.VMEM((4, _B * _W, 128), jnp.float32)],
        compiler_params=pltpu.CompilerParams(
            dimension_semantics=("parallel",),
            flags={"XLA_TPU_STORE_TO_LOAD_FORWARDING_WINDOW": 24576},
        ),
        name="hex_max_pool",
    )(xf, s1, s0, s2)

    return out.reshape(*shape_bac, _HO, _HO)


# speedup vs baseline: 1.0410x; 1.0006x over previous
import jax
import jax.numpy as jnp
from jax.experimental import pallas as pl
from jax.experimental.pallas import tpu as pltpu

_PAD = 1e-9
_W = 512
_HO = 256
_B = 8


def _pool_body(x_ref, s1_ref, s0_ref, s2_ref, o_ref, sc_ref):
    for k in range(4):
        sc_ref[k] = x_ref[:, 128 * k:128 * (k + 1)]

    io = jax.lax.broadcasted_iota(jnp.int32, (_HO, _W), 0)
    jj = jax.lax.broadcasted_iota(jnp.int32, (_HO, _W), 1)
    cond_e = 2 * io + jj == _W
    cond_o = 2 * io + 1 + jj == _W
    jo = jax.lax.broadcasted_iota(jnp.int32, (_HO, _HO), 1)
    iom = jax.lax.broadcasted_iota(jnp.int32, (_HO, _HO), 0)

    for b in range(_B):
        base = b * _W
        xe = jnp.concatenate(
            [sc_ref[pl.ds(k, 1), pl.ds(base, _HO, 2), :][0]
             for k in range(4)], axis=1)
        xo = jnp.concatenate(
            [sc_ref[pl.ds(k, 1), pl.ds(base + 1, _HO, 2), :][0]
             for k in range(4)], axis=1)

        xe = jnp.where(cond_e, _PAD, xe)
        xo = jnp.where(cond_o, _PAD, xo)

        a = jnp.concatenate(
            [jnp.full((1, _W), _PAD, jnp.float32), xo[:-1, :]], axis=0)

        m1 = jnp.maximum(xe, xo)
        m0 = jnp.maximum(a, m1)
        m2 = jnp.maximum(a, xe)

        out = jnp.dot(m0, s0_ref[...], preferred_element_type=jnp.float32)
        out = jnp.maximum(out, jnp.dot(m1, s1_ref[...],
                                       preferred_element_type=jnp.float32))
        out = jnp.maximum(out, jnp.dot(m2, s2_ref[...],
                                       preferred_element_type=jnp.float32))

        out = jnp.where(jo == 0, jnp.maximum(out, _PAD), out)

        o_ref[b * _HO:(b + 1) * _HO, :] = jnp.where(iom + jo < _HO, out, 0.0)


@jax.jit
def kernel(x):
    shape_bac = x.shape[:-2]
    n = 1
    for d in shape_bac:
        n *= d
    xf = x.reshape(n * _W, _W)

    c = jnp.arange(_W, dtype=jnp.int32)[:, None]
    oj2 = 2 * jnp.arange(_HO, dtype=jnp.int32)[None, :]
    s1 = (c == oj2 - 1).astype(jnp.float32)
    s0 = (c == oj2).astype(jnp.float32)
    s2 = (c == oj2 + 1).astype(jnp.float32)

    sel_spec = pl.BlockSpec((_W, _HO), lambda i: (0, 0))
    out = pl.pallas_call(
        _pool_body,
        grid=(n // _B,),
        in_specs=[
            pl.BlockSpec((_B * _W, _W), lambda i: (i, 0)),
            sel_spec, sel_spec, sel_spec,
        ],
        out_specs=pl.BlockSpec((_B * _HO, _HO), lambda i: (i, 0)),
        out_shape=jax.ShapeDtypeStruct((n * _HO, _HO), jnp.float32),
        scratch_shapes=[pltpu.VMEM((4, _B * _W, 128), jnp.float32)],
        compiler_params=pltpu.CompilerParams(
            dimension_semantics=("parallel",),
            flags={"XLA_TPU_STORE_TO_LOAD_FORWARDING_WINDOW": 49152},
        ),
        name="hex_max_pool",
    )(xf, s1, s0, s2)

    return out.reshape(*shape_bac, _HO, _HO)


# speedup vs baseline: 1.1091x; 1.0655x over previous
import jax
import jax.numpy as jnp
from jax.experimental import pallas as pl
from jax.experimental.pallas import tpu as pltpu

_PAD = 1e-9
_W = 512
_HO = 256
_B = 16


def _pool_body(x_ref, s1_ref, s0_ref, s2_ref, o_ref, sc_ref):
    io = jax.lax.broadcasted_iota(jnp.int32, (_HO, _W), 0)
    jj = jax.lax.broadcasted_iota(jnp.int32, (_HO, _W), 1)
    cond_e = 2 * io + jj == _W
    cond_o = 2 * io + 1 + jj == _W
    jo = jax.lax.broadcasted_iota(jnp.int32, (_HO, _HO), 1)
    iom = jax.lax.broadcasted_iota(jnp.int32, (_HO, _HO), 0)

    for b in range(_B):
        half = b % (_B // 2)
        if half == 0:
            off = (b // (_B // 2)) * (_B // 2) * _W
            for k in range(4):
                sc_ref[k] = x_ref[off:off + (_B // 2) * _W,
                                  128 * k:128 * (k + 1)]
        base = half * _W
        xe = jnp.concatenate(
            [sc_ref[pl.ds(k, 1), pl.ds(base, _HO, 2), :][0]
             for k in range(4)], axis=1)
        xo = jnp.concatenate(
            [sc_ref[pl.ds(k, 1), pl.ds(base + 1, _HO, 2), :][0]
             for k in range(4)], axis=1)

        xe = jnp.where(cond_e, _PAD, xe)
        xo = jnp.where(cond_o, _PAD, xo)

        a = jnp.concatenate(
            [jnp.full((1, _W), _PAD, jnp.float32), xo[:-1, :]], axis=0)

        m1 = jnp.maximum(xe, xo)
        m0 = jnp.maximum(a, m1)
        m2 = jnp.maximum(a, xe)

        out = jnp.dot(m0, s0_ref[...], preferred_element_type=jnp.float32)
        out = jnp.maximum(out, jnp.dot(m1, s1_ref[...],
                                       preferred_element_type=jnp.float32))
        out = jnp.maximum(out, jnp.dot(m2, s2_ref[...],
                                       preferred_element_type=jnp.float32))

        out = jnp.where(jo == 0, jnp.maximum(out, _PAD), out)

        o_ref[b * _HO:(b + 1) * _HO, :] = jnp.where(iom + jo < _HO, out, 0.0)


@jax.jit
def kernel(x):
    shape_bac = x.shape[:-2]
    n = 1
    for d in shape_bac:
        n *= d
    xf = x.reshape(n * _W, _W)

    c = jnp.arange(_W, dtype=jnp.int32)[:, None]
    oj2 = 2 * jnp.arange(_HO, dtype=jnp.int32)[None, :]
    s1 = (c == oj2 - 1).astype(jnp.float32)
    s0 = (c == oj2).astype(jnp.float32)
    s2 = (c == oj2 + 1).astype(jnp.float32)

    sel_spec = pl.BlockSpec((_W, _HO), lambda i: (0, 0))
    out = pl.pallas_call(
        _pool_body,
        grid=(n // _B,),
        in_specs=[
            pl.BlockSpec((_B * _W, _W), lambda i: (i, 0)),
            sel_spec, sel_spec, sel_spec,
        ],
        out_specs=pl.BlockSpec((_B * _HO, _HO), lambda i: (i, 0)),
        out_shape=jax.ShapeDtypeStruct((n * _HO, _HO), jnp.float32),
        scratch_shapes=[pltpu.VMEM((4, (_B // 2) * _W, 128), jnp.float32)],
        compiler_params=pltpu.CompilerParams(
            dimension_semantics=("parallel",),
            flags={"XLA_TPU_STORE_TO_LOAD_FORWARDING_WINDOW": 49152},
            vmem_limit_bytes=56 * 1024 * 1024,
        ),
        name="hex_max_pool",
    )(xf, s1, s0, s2)

    return out.reshape(*shape_bac, _HO, _HO)
